# Initial kernel scaffold; baseline (speedup 1.0000x reference)
import functools

import jax
import jax.numpy as jnp
from jax import lax
from jax.experimental import pallas as pl
from jax.experimental.pallas import tpu as pltpu
from jax.experimental.pallas import tpu_sc as plsc

_K = 5
_LEAKY = 0.2
_EPS = 1e-5


def _prep_body(xb_ref, wat_ref, wdt_ref, gidx_ref, ya_ref, c_ref, *, rb, n):
    b = pl.program_id(0)
    r = pl.program_id(1)
    xb = xb_ref[0]
    xr = xb_ref[0, pl.ds(r * rb, rb), :]
    x2b = jnp.sum(xb * xb, axis=1).reshape(1, n)
    x2r = jnp.sum(xr * xr, axis=1, keepdims=True)
    gram = lax.dot_general(xr, xb, (((1,), (1,)), ((), ())),
                           preferred_element_type=jnp.float32)
    dist = 2.0 * gram - x2r - x2b
    cols = lax.broadcasted_iota(jnp.int32, (rb, n), 1)
    base = b * n
    for j in range(_K):
        m = jnp.max(dist, axis=1, keepdims=True)
        i = jnp.min(jnp.where(dist == m, cols, n), axis=1)
        gidx_ref[0, j, :] = i + base
        dist = jnp.where(cols == i[:, None], -jnp.inf, dist)
    ya_ref[0] = jnp.dot(xr, wat_ref[...], preferred_element_type=jnp.float32)
    c_ref[0] = jnp.dot(xr, wdt_ref[...], preferred_element_type=jnp.float32)


def _prep(x, wat, wdt, rb=256):
    B, N, C = x.shape
    O = wat.shape[1]
    out_shape = [
        jax.ShapeDtypeStruct((B, _K, N), jnp.int32),
        jax.ShapeDtypeStruct((B, N, O), jnp.float32),
        jax.ShapeDtypeStruct((B, N, O), jnp.float32),
    ]
    return pl.pallas_call(
        functools.partial(_prep_body, rb=rb, n=N),
        grid=(B, N // rb),
        in_specs=[
            pl.BlockSpec((1, N, C), lambda b, r: (b, 0, 0)),
            pl.BlockSpec((C, O), lambda b, r: (0, 0)),
            pl.BlockSpec((C, O), lambda b, r: (0, 0)),
        ],
        out_specs=[
            pl.BlockSpec((1, _K, rb), lambda b, r: (b, 0, r)),
            pl.BlockSpec((1, rb, O), lambda b, r: (b, r, 0)),
            pl.BlockSpec((1, rb, O), lambda b, r: (b, r, 0)),
        ],
        out_shape=out_shape,
    )(x, wat, wdt)


def _sc_gather(ya, c, gidx, cp=16):
    BN, O = ya.shape
    nw = 32
    P = BN // nw
    nchunk = P // cp
    cp5 = cp * _K
    nsl = O // 16
    mesh = plsc.VectorSubcoreMesh(core_axis_name="c", subcore_axis_name="s")

    def body(ya_hbm, c_hbm, gidx_hbm, zmax_hbm, zmin_hbm, ps_hbm, ps2_hbm,
             idxbuf, rows, cbuf, zmaxbuf, zminbuf, accs, accs2, sem):
        wid = lax.axis_index("s") * 2 + lax.axis_index("c")
        base = wid * P

        def zinit(s, carry):
            sl = pl.ds(s * 16, 16)
            accs[0, sl] = jnp.zeros((16,), jnp.float32)
            accs2[0, sl] = jnp.zeros((16,), jnp.float32)
            return carry
        lax.fori_loop(0, nsl, zinit, 0)

        def chunk(ch, carry):
            pt0 = base + ch * cp
            pltpu.sync_copy(gidx_hbm.at[pl.ds(pt0 * _K, cp5)], idxbuf)
            pltpu.async_copy(ya_hbm.at[idxbuf], rows, sem).wait()
            pltpu.sync_copy(c_hbm.at[pl.ds(pt0, cp)], cbuf)

            def sbody(s, carry2):
                sl = pl.ds(s * 16, 16)
                acc = jnp.zeros((16,), jnp.float32)
                acc2 = jnp.zeros((16,), jnp.float32)
                for p in range(cp):
                    cv = cbuf[p, sl]
                    z0 = rows[_K * p + 0, sl] + cv
                    z1 = rows[_K * p + 1, sl] + cv
                    z2 = rows[_K * p + 2, sl] + cv
                    z3 = rows[_K * p + 3, sl] + cv
                    z4 = rows[_K * p + 4, sl] + cv
                    zmaxbuf[p, sl] = jnp.maximum(
                        jnp.maximum(jnp.maximum(z0, z1), jnp.maximum(z2, z3)), z4)
                    zminbuf[p, sl] = jnp.minimum(
                        jnp.minimum(jnp.minimum(z0, z1), jnp.minimum(z2, z3)), z4)
                    acc = acc + ((z0 + z1) + (z2 + z3) + z4)
                    acc2 = acc2 + ((z0 * z0 + z1 * z1) + (z2 * z2 + z3 * z3)
                                   + z4 * z4)
                accs[0, sl] = accs[0, sl] + acc
                accs2[0, sl] = accs2[0, sl] + acc2
                return carry2
            lax.fori_loop(0, nsl, sbody, 0)
            pltpu.sync_copy(zmaxbuf, zmax_hbm.at[pl.ds(pt0, cp)])
            pltpu.sync_copy(zminbuf, zmin_hbm.at[pl.ds(pt0, cp)])
            return carry
        lax.fori_loop(0, nchunk, chunk, 0)
        pltpu.sync_copy(accs, ps_hbm.at[pl.ds(wid, 1)])
        pltpu.sync_copy(accs2, ps2_hbm.at[pl.ds(wid, 1)])

    f = pl.kernel(
        body,
        out_type=[
            jax.ShapeDtypeStruct((BN, O), jnp.float32),
            jax.ShapeDtypeStruct((BN, O), jnp.float32),
            jax.ShapeDtypeStruct((nw, O), jnp.float32),
            jax.ShapeDtypeStruct((nw, O), jnp.float32),
        ],
        mesh=mesh,
        scratch_types=[
            pltpu.VMEM((cp5,), jnp.int32),
            pltpu.VMEM((cp5, O), jnp.float32),
            pltpu.VMEM((cp, O), jnp.float32),
            pltpu.VMEM((cp, O), jnp.float32),
            pltpu.VMEM((cp, O), jnp.float32),
            pltpu.VMEM((1, O), jnp.float32),
            pltpu.VMEM((1, O), jnp.float32),
            pltpu.SemaphoreType.DMA,
        ],
    )
    return f(ya, c, gidx)


def _fin_body(zmax_ref, zmin_ref, ps_ref, ps2_ref, g_ref, bta_ref, out_ref,
              *, cnt):
    s1 = jnp.sum(ps_ref[...], axis=0, keepdims=True)
    s2 = jnp.sum(ps2_ref[...], axis=0, keepdims=True)
    mean = s1 * (1.0 / cnt)
    var = s2 * (1.0 / cnt) - mean * mean
    inv = lax.rsqrt(var + _EPS)
    gam = g_ref[...]
    scale = gam * inv
    off = bta_ref[...] - mean * scale
    sel = jnp.where(gam >= 0.0, zmax_ref[...], zmin_ref[...])
    t = sel * scale + off
    out_ref[...] = jnp.where(t >= 0.0, t, _LEAKY * t)


def _fin(zmax, zmin, ps, ps2, g, bta, cnt, rb=512):
    BN, O = zmax.shape
    nw = ps.shape[0]
    return pl.pallas_call(
        functools.partial(_fin_body, cnt=float(cnt)),
        grid=(BN // rb,),
        in_specs=[
            pl.BlockSpec((rb, O), lambda r: (r, 0)),
            pl.BlockSpec((rb, O), lambda r: (r, 0)),
            pl.BlockSpec((nw, O), lambda r: (0, 0)),
            pl.BlockSpec((nw, O), lambda r: (0, 0)),
            pl.BlockSpec((1, O), lambda r: (0, 0)),
            pl.BlockSpec((1, O), lambda r: (0, 0)),
        ],
        out_specs=pl.BlockSpec((rb, O), lambda r: (r, 0)),
        out_shape=jax.ShapeDtypeStruct((BN, O), jnp.float32),
    )(zmax, zmin, ps, ps2, g, bta)


def _layer(x, W, g, bta):
    B, N, C = x.shape
    O = W.shape[0]
    wat = jnp.transpose(W[:, :C])
    wdt = jnp.transpose(W[:, C:] - W[:, :C])
    gidx, ya, c = _prep(x, wat, wdt)
    gflat = jnp.transpose(gidx, (0, 2, 1)).reshape(-1)
    zmax, zmin, ps, ps2 = _sc_gather(
        ya.reshape(B * N, O), c.reshape(B * N, O), gflat)
    out = _fin(zmax, zmin, ps, ps2, g.reshape(1, O), bta.reshape(1, O),
               B * N * _K)
    return out.reshape(B, N, O)


def kernel(interm_repr, W1, g1, b1, W2, g2, b2):
    x1 = _layer(interm_repr, W1, g1, b1)
    return _layer(x1, W2, g2, b2)


# speedup vs baseline: 11.8177x; 11.8177x over previous
import functools

import jax
import jax.numpy as jnp
from jax import lax
from jax.experimental import pallas as pl
from jax.experimental.pallas import tpu as pltpu
from jax.experimental.pallas import tpu_sc as plsc

_K = 5
_LEAKY = 0.2
_EPS = 1e-5


def _prep_body(xb_ref, wbt_ref, gidx_ref, cb_ref, *, rb, n):
    b = pl.program_id(0)
    r = pl.program_id(1)
    xb = xb_ref[0]
    xr = xb_ref[0, pl.ds(r * rb, rb), :]
    x2b = jnp.sum(xb * xb, axis=1).reshape(1, n)
    x2r = jnp.sum(xr * xr, axis=1, keepdims=True)
    gram = lax.dot_general(xr, xb, (((1,), (1,)), ((), ())),
                           preferred_element_type=jnp.float32)
    dist = -(x2r + x2b - 2.0 * gram)
    cols = lax.broadcasted_iota(jnp.int32, (rb, n), 1)
    base = b * n
    for j in range(_K):
        m = jnp.max(dist, axis=1, keepdims=True)
        i = jnp.min(jnp.where(dist == m, cols, n), axis=1)
        gidx_ref[0, j, :] = i + base
        dist = jnp.where(cols == i[:, None], -jnp.inf, dist)
    cb_ref[0] = jnp.dot(xr, wbt_ref[...], preferred_element_type=jnp.float32)


def _prep(x, wbt, rb=256):
    B, N, C = x.shape
    O = wbt.shape[1]
    out_shape = [
        jax.ShapeDtypeStruct((B, _K, N), jnp.int32),
        jax.ShapeDtypeStruct((B, N, O), jnp.float32),
    ]
    return pl.pallas_call(
        functools.partial(_prep_body, rb=rb, n=N),
        grid=(B, N // rb),
        in_specs=[
            pl.BlockSpec((1, N, C), lambda b, r: (b, 0, 0)),
            pl.BlockSpec((C, O), lambda b, r: (0, 0)),
        ],
        out_specs=[
            pl.BlockSpec((1, _K, rb), lambda b, r: (b, 0, r)),
            pl.BlockSpec((1, rb, O), lambda b, r: (b, r, 0)),
        ],
        out_shape=out_shape,
    )(x, wbt)


def _sc_gather(xf, gidx, cp=16):
    BN, C = xf.shape
    nw = 32
    P = BN // nw
    nchunk = P // cp
    cp5 = cp * _K
    mesh = plsc.VectorSubcoreMesh(core_axis_name="c", subcore_axis_name="s")

    def body(xf_hbm, gidx_hbm, feat_hbm, idxbuf, rows, sem):
        wid = lax.axis_index("s") * 2 + lax.axis_index("c")
        base = wid * P

        def chunk(ch, carry):
            pt0 = base + ch * cp
            pltpu.sync_copy(gidx_hbm.at[pl.ds(pt0 * _K, cp5)], idxbuf)
            pltpu.async_copy(xf_hbm.at[idxbuf], rows, sem).wait()
            pltpu.sync_copy(rows, feat_hbm.at[pl.ds(pt0 * _K, cp5)])
            return carry
        lax.fori_loop(0, nchunk, chunk, 0)

    f = pl.kernel(
        body,
        out_type=jax.ShapeDtypeStruct((BN * _K, C), jnp.float32),
        mesh=mesh,
        scratch_types=[
            pltpu.VMEM((cp5,), jnp.int32),
            pltpu.VMEM((cp5, C), jnp.float32),
            pltpu.SemaphoreType.DMA,
        ],
    )
    return f(xf, gidx)


def _conv_body(feat_ref, x_ref, cb_ref, wat_ref, zmax_ref, zmin_ref,
               ps_ref, ps2_ref):
    xc = x_ref[...]
    cbv = cb_ref[...]
    wat = wat_ref[...]
    zm = zn = acc = acc2 = None
    for j in range(_K):
        d = feat_ref[:, j, :] - xc
        z = jnp.dot(d, wat, preferred_element_type=jnp.float32) + cbv
        if j == 0:
            zm, zn, acc, acc2 = z, z, z, z * z
        else:
            zm = jnp.maximum(zm, z)
            zn = jnp.minimum(zn, z)
            acc = acc + z
            acc2 = acc2 + z * z
    zmax_ref[...] = zm
    zmin_ref[...] = zn
    ps_ref[0] = jnp.sum(acc, axis=0, keepdims=True)
    ps2_ref[0] = jnp.sum(acc2, axis=0, keepdims=True)


def _conv(feat, xf, cb, wat, rb=256):
    BN, C = xf.shape
    O = wat.shape[1]
    G = BN // rb
    out_shape = [
        jax.ShapeDtypeStruct((BN, O), jnp.float32),
        jax.ShapeDtypeStruct((BN, O), jnp.float32),
        jax.ShapeDtypeStruct((G, 1, O), jnp.float32),
        jax.ShapeDtypeStruct((G, 1, O), jnp.float32),
    ]
    zmax, zmin, ps, ps2 = pl.pallas_call(
        _conv_body,
        grid=(G,),
        in_specs=[
            pl.BlockSpec((rb, _K, C), lambda g: (g, 0, 0)),
            pl.BlockSpec((rb, C), lambda g: (g, 0)),
            pl.BlockSpec((rb, O), lambda g: (g, 0)),
            pl.BlockSpec((C, O), lambda g: (0, 0)),
        ],
        out_specs=[
            pl.BlockSpec((rb, O), lambda g: (g, 0)),
            pl.BlockSpec((rb, O), lambda g: (g, 0)),
            pl.BlockSpec((1, 1, O), lambda g: (g, 0, 0)),
            pl.BlockSpec((1, 1, O), lambda g: (g, 0, 0)),
        ],
        out_shape=out_shape,
    )(feat.reshape(BN, _K, C), xf, cb, wat)
    return zmax, zmin, ps.reshape(G, O), ps2.reshape(G, O)


def _fin_body(zmax_ref, zmin_ref, ps_ref, ps2_ref, g_ref, bta_ref, out_ref,
              *, cnt):
    s1 = jnp.sum(ps_ref[...], axis=0, keepdims=True)
    s2 = jnp.sum(ps2_ref[...], axis=0, keepdims=True)
    mean = s1 * (1.0 / cnt)
    var = s2 * (1.0 / cnt) - mean * mean
    inv = lax.rsqrt(var + _EPS)
    gam = g_ref[...]
    scale = gam * inv
    off = bta_ref[...] - mean * scale
    sel = jnp.where(gam >= 0.0, zmax_ref[...], zmin_ref[...])
    t = sel * scale + off
    out_ref[...] = jnp.where(t >= 0.0, t, _LEAKY * t)


def _fin(zmax, zmin, ps, ps2, g, bta, cnt, rb=512):
    BN, O = zmax.shape
    G = ps.shape[0]
    return pl.pallas_call(
        functools.partial(_fin_body, cnt=float(cnt)),
        grid=(BN // rb,),
        in_specs=[
            pl.BlockSpec((rb, O), lambda r: (r, 0)),
            pl.BlockSpec((rb, O), lambda r: (r, 0)),
            pl.BlockSpec((G, O), lambda r: (0, 0)),
            pl.BlockSpec((G, O), lambda r: (0, 0)),
            pl.BlockSpec((1, O), lambda r: (0, 0)),
            pl.BlockSpec((1, O), lambda r: (0, 0)),
        ],
        out_specs=pl.BlockSpec((rb, O), lambda r: (r, 0)),
        out_shape=jax.ShapeDtypeStruct((BN, O), jnp.float32),
    )(zmax, zmin, ps, ps2, g, bta)


def _layer(x, W, g, bta):
    B, N, C = x.shape
    O = W.shape[0]
    wat = jnp.transpose(W[:, :C])
    wbt = jnp.transpose(W[:, C:])
    gidx, cb = _prep(x, wbt)
    gflat = jnp.transpose(gidx, (0, 2, 1)).reshape(-1)
    xf = x.reshape(B * N, C)
    feat = _sc_gather(xf, gflat)
    zmax, zmin, ps, ps2 = _conv(feat, xf, cb.reshape(B * N, O), wat)
    out = _fin(zmax, zmin, ps, ps2, g.reshape(1, O), bta.reshape(1, O),
               B * N * _K)
    return out.reshape(B, N, O)


def kernel(interm_repr, W1, g1, b1, W2, g2, b2):
    x1 = _layer(interm_repr, W1, g1, b1)
    return _layer(x1, W2, g2, b2)


# speedup vs baseline: 16.7396x; 1.4165x over previous
import functools

import jax
import jax.numpy as jnp
from jax import lax
from jax.experimental import pallas as pl
from jax.experimental.pallas import tpu as pltpu
from jax.experimental.pallas import tpu_sc as plsc

_K = 5
_LEAKY = 0.2
_EPS = 1e-5


def _prep_body(xb_ref, wbt_ref, gidx_ref, cb_ref, *, rb, n):
    b = pl.program_id(0)
    r = pl.program_id(1)
    xb = xb_ref[0]
    xr = xb_ref[0, pl.ds(r * rb, rb), :]
    x2b = jnp.sum(xb * xb, axis=1).reshape(1, n)
    x2r = jnp.sum(xr * xr, axis=1, keepdims=True)
    gram = lax.dot_general(xr, xb, (((1,), (1,)), ((), ())),
                           preferred_element_type=jnp.float32)
    dist = -(x2r + x2b - 2.0 * gram)
    colsf = lax.broadcasted_iota(jnp.int32, (rb, n), 1).astype(jnp.float32)
    base = b * n
    for j in range(_K):
        m = jnp.max(dist, axis=1, keepdims=True)
        fi = jnp.min(jnp.where(dist == m, colsf, float(n)), axis=1)
        gidx_ref[0, j, :] = fi.astype(jnp.int32) + base
        dist = jnp.where(colsf == fi[:, None], -jnp.inf, dist)
    cb_ref[0] = jnp.dot(xr, wbt_ref[...], preferred_element_type=jnp.float32)


def _prep(x, wbt, rb=256):
    B, N, C = x.shape
    O = wbt.shape[1]
    out_shape = [
        jax.ShapeDtypeStruct((B, _K, N), jnp.int32),
        jax.ShapeDtypeStruct((B, N, O), jnp.float32),
    ]
    return pl.pallas_call(
        functools.partial(_prep_body, rb=rb, n=N),
        grid=(B, N // rb),
        in_specs=[
            pl.BlockSpec((1, N, C), lambda b, r: (b, 0, 0)),
            pl.BlockSpec((C, O), lambda b, r: (0, 0)),
        ],
        out_specs=[
            pl.BlockSpec((1, _K, rb), lambda b, r: (b, 0, r)),
            pl.BlockSpec((1, rb, O), lambda b, r: (b, r, 0)),
        ],
        out_shape=out_shape,
    )(x, wbt)


def _sc_gather(xf, gidxj, cp=128):
    BN, C = xf.shape
    nw = 32
    P = BN // nw
    nstep = _K * (P // cp)
    mesh = plsc.VectorSubcoreMesh(core_axis_name="c", subcore_axis_name="s")

    def body(xf_hbm, gidxj_hbm, feat_hbm, idx0, idx1, rows0, rows1,
             sem0, sem1):
        wid = lax.axis_index("s") * 2 + lax.axis_index("c")
        base = wid * P
        idxb = (idx0, idx1)
        rowsb = (rows0, rows1)
        semb = (sem0, sem1)

        def off(t):
            j, h = t // (P // cp), t % (P // cp)
            return j * BN + base + h * cp

        pltpu.sync_copy(gidxj_hbm.at[pl.ds(off(0), cp)], idxb[0])
        cps = [pltpu.async_copy(xf_hbm.at[idxb[0]], rowsb[0], semb[0])]
        for t in range(nstep):
            pa = t % 2
            if t + 1 < nstep:
                pb = (t + 1) % 2
                pltpu.sync_copy(gidxj_hbm.at[pl.ds(off(t + 1), cp)], idxb[pb])
                cps.append(
                    pltpu.async_copy(xf_hbm.at[idxb[pb]], rowsb[pb], semb[pb]))
            cps[t].wait()
            pltpu.sync_copy(rowsb[pa], feat_hbm.at[pl.ds(off(t), cp)])

    f = pl.kernel(
        body,
        out_type=jax.ShapeDtypeStruct((_K * BN, C), jnp.float32),
        mesh=mesh,
        scratch_types=[
            pltpu.VMEM((cp,), jnp.int32),
            pltpu.VMEM((cp,), jnp.int32),
            pltpu.VMEM((cp, C), jnp.float32),
            pltpu.VMEM((cp, C), jnp.float32),
            pltpu.SemaphoreType.DMA,
            pltpu.SemaphoreType.DMA,
        ],
    )
    return f(xf, gidxj)


def _conv_body(feat_ref, x_ref, cb_ref, wat_ref, zmax_ref, zmin_ref,
               ps_ref, ps2_ref):
    xc = x_ref[...]
    cbv = cb_ref[...]
    wat = wat_ref[...]
    zm = zn = acc = acc2 = None
    for j in range(_K):
        d = feat_ref[j] - xc
        z = jnp.dot(d, wat, preferred_element_type=jnp.float32) + cbv
        if j == 0:
            zm, zn, acc, acc2 = z, z, z, z * z
        else:
            zm = jnp.maximum(zm, z)
            zn = jnp.minimum(zn, z)
            acc = acc + z
            acc2 = acc2 + z * z
    zmax_ref[...] = zm
    zmin_ref[...] = zn
    ps_ref[0] = jnp.sum(acc, axis=0, keepdims=True)
    ps2_ref[0] = jnp.sum(acc2, axis=0, keepdims=True)


def _conv(feat, xf, cb, wat, rb=256):
    BN, C = xf.shape
    O = wat.shape[1]
    G = BN // rb
    out_shape = [
        jax.ShapeDtypeStruct((BN, O), jnp.float32),
        jax.ShapeDtypeStruct((BN, O), jnp.float32),
        jax.ShapeDtypeStruct((G, 1, O), jnp.float32),
        jax.ShapeDtypeStruct((G, 1, O), jnp.float32),
    ]
    zmax, zmin, ps, ps2 = pl.pallas_call(
        _conv_body,
        grid=(G,),
        in_specs=[
            pl.BlockSpec((_K, rb, C), lambda g: (0, g, 0)),
            pl.BlockSpec((rb, C), lambda g: (g, 0)),
            pl.BlockSpec((rb, O), lambda g: (g, 0)),
            pl.BlockSpec((C, O), lambda g: (0, 0)),
        ],
        out_specs=[
            pl.BlockSpec((rb, O), lambda g: (g, 0)),
            pl.BlockSpec((rb, O), lambda g: (g, 0)),
            pl.BlockSpec((1, 1, O), lambda g: (g, 0, 0)),
            pl.BlockSpec((1, 1, O), lambda g: (g, 0, 0)),
        ],
        out_shape=out_shape,
    )(feat.reshape(_K, BN, C), xf, cb, wat)
    return zmax, zmin, ps.reshape(G, O), ps2.reshape(G, O)


def _fin_body(zmax_ref, zmin_ref, ps_ref, ps2_ref, g_ref, bta_ref, out_ref,
              *, cnt):
    s1 = jnp.sum(ps_ref[...], axis=0, keepdims=True)
    s2 = jnp.sum(ps2_ref[...], axis=0, keepdims=True)
    mean = s1 * (1.0 / cnt)
    var = s2 * (1.0 / cnt) - mean * mean
    inv = lax.rsqrt(var + _EPS)
    gam = g_ref[...]
    scale = gam * inv
    off = bta_ref[...] - mean * scale
    sel = jnp.where(gam >= 0.0, zmax_ref[...], zmin_ref[...])
    t = sel * scale + off
    out_ref[...] = jnp.where(t >= 0.0, t, _LEAKY * t)


def _fin(zmax, zmin, ps, ps2, g, bta, cnt, rb=512):
    BN, O = zmax.shape
    G = ps.shape[0]
    return pl.pallas_call(
        functools.partial(_fin_body, cnt=float(cnt)),
        grid=(BN // rb,),
        in_specs=[
            pl.BlockSpec((rb, O), lambda r: (r, 0)),
            pl.BlockSpec((rb, O), lambda r: (r, 0)),
            pl.BlockSpec((G, O), lambda r: (0, 0)),
            pl.BlockSpec((G, O), lambda r: (0, 0)),
            pl.BlockSpec((1, O), lambda r: (0, 0)),
            pl.BlockSpec((1, O), lambda r: (0, 0)),
        ],
        out_specs=pl.BlockSpec((rb, O), lambda r: (r, 0)),
        out_shape=jax.ShapeDtypeStruct((BN, O), jnp.float32),
    )(zmax, zmin, ps, ps2, g, bta)


def _layer(x, W, g, bta):
    B, N, C = x.shape
    O = W.shape[0]
    wat = jnp.transpose(W[:, :C])
    wbt = jnp.transpose(W[:, C:])
    gidx, cb = _prep(x, wbt)
    gflatj = jnp.transpose(gidx, (1, 0, 2)).reshape(-1)
    xf = x.reshape(B * N, C)
    feat = _sc_gather(xf, gflatj)
    zmax, zmin, ps, ps2 = _conv(feat, xf, cb.reshape(B * N, O), wat)
    out = _fin(zmax, zmin, ps, ps2, g.reshape(1, O), bta.reshape(1, O),
               B * N * _K)
    return out.reshape(B, N, O)


def kernel(interm_repr, W1, g1, b1, W2, g2, b2):
    x1 = _layer(interm_repr, W1, g1, b1)
    return _layer(x1, W2, g2, b2)


# speedup vs baseline: 16.7462x; 1.0004x over previous
import functools

import jax
import jax.numpy as jnp
from jax import lax
from jax.experimental import pallas as pl
from jax.experimental.pallas import tpu as pltpu
from jax.experimental.pallas import tpu_sc as plsc

_K = 5
_LEAKY = 0.2
_EPS = 1e-5


def _prep_body(xb_ref, wb_ref, gidx_ref, cb_ref, *, rb, n):
    b = pl.program_id(0)
    r = pl.program_id(1)
    xb = xb_ref[0]
    xr = xb_ref[0, pl.ds(r * rb, rb), :]
    x2b = jnp.sum(xb * xb, axis=1).reshape(1, n)
    x2r = jnp.sum(xr * xr, axis=1, keepdims=True)
    gram = lax.dot_general(xr, xb, (((1,), (1,)), ((), ())),
                           preferred_element_type=jnp.float32)
    dist = -(x2r + x2b - 2.0 * gram)
    colsf = lax.broadcasted_iota(jnp.int32, (rb, n), 1).astype(jnp.float32)
    base = b * n
    for j in range(_K):
        m = jnp.max(dist, axis=1, keepdims=True)
        fi = jnp.min(jnp.where(dist == m, colsf, float(n)), axis=1)
        gidx_ref[j, 0, 0, :] = fi.astype(jnp.int32) + base
        if j + 1 < _K:
            dist = jnp.where(colsf == fi[:, None], -jnp.inf, dist)
    cb_ref[0] = lax.dot_general(xr, wb_ref[...], (((1,), (1,)), ((), ())),
                                preferred_element_type=jnp.float32)


def _prep(x, W, rb=256):
    B, N, C = x.shape
    O = W.shape[0]
    out_shape = [
        jax.ShapeDtypeStruct((_K, B, 1, N), jnp.int32),
        jax.ShapeDtypeStruct((B, N, O), jnp.float32),
    ]
    return pl.pallas_call(
        functools.partial(_prep_body, rb=rb, n=N),
        grid=(B, N // rb),
        in_specs=[
            pl.BlockSpec((1, N, C), lambda b, r: (b, 0, 0)),
            pl.BlockSpec((O, C), lambda b, r: (0, 1)),
        ],
        out_specs=[
            pl.BlockSpec((_K, 1, 1, rb), lambda b, r: (0, b, 0, r)),
            pl.BlockSpec((1, rb, O), lambda b, r: (b, r, 0)),
        ],
        out_shape=out_shape,
    )(x, W)


def _sc_gather(xf, gidxj, cp=128):
    BN, C = xf.shape
    nw = 32
    P = BN // nw
    nstep = _K * (P // cp)
    mesh = plsc.VectorSubcoreMesh(core_axis_name="c", subcore_axis_name="s")

    nh = P // cp

    def body(xf_hbm, gidxj_hbm, feat_hbm, slab, rows0, rows1, rows2,
             gs0, gs1, gs2, ws0, ws1, ws2):
        wid = lax.axis_index("s") * 2 + lax.axis_index("c")
        base = wid * P
        rowsb = (rows0, rows1, rows2)
        gsem = (gs0, gs1, gs2)
        wsem = (ws0, ws1, ws2)

        for j in range(_K):
            pltpu.sync_copy(gidxj_hbm.at[pl.ds(j * BN + base, P)],
                            slab.at[pl.ds(j * P, P)])

        def off(t):
            j, h = divmod(t, nh)
            return j * BN + base + h * cp

        def idxsl(t):
            j, h = divmod(t, nh)
            return slab.at[pl.ds(j * P + h * cp, cp)]

        g = [None] * nstep
        w = [None] * nstep
        for s in range(min(3, nstep)):
            g[s] = pltpu.async_copy(xf_hbm.at[idxsl(s)], rowsb[s % 3],
                                    gsem[s % 3])
        for t in range(nstep):
            p = t % 3
            g[t].wait()
            w[t] = pltpu.async_copy(rowsb[p], feat_hbm.at[pl.ds(off(t), cp)],
                                    wsem[p])
            if t + 3 < nstep:
                w[t].wait()
                g[t + 3] = pltpu.async_copy(xf_hbm.at[idxsl(t + 3)], rowsb[p],
                                            gsem[p])
        for t in range(max(0, nstep - 3), nstep):
            w[t].wait()

    f = pl.kernel(
        body,
        out_type=jax.ShapeDtypeStruct((_K * BN, C), jnp.float32),
        mesh=mesh,
        scratch_types=[
            pltpu.VMEM((_K * P,), jnp.int32),
            pltpu.VMEM((cp, C), jnp.float32),
            pltpu.VMEM((cp, C), jnp.float32),
            pltpu.VMEM((cp, C), jnp.float32),
            pltpu.SemaphoreType.DMA,
            pltpu.SemaphoreType.DMA,
            pltpu.SemaphoreType.DMA,
            pltpu.SemaphoreType.DMA,
            pltpu.SemaphoreType.DMA,
            pltpu.SemaphoreType.DMA,
        ],
    )
    return f(xf, gidxj)


def _conv_body(feat_ref, x_ref, cb_ref, wa_ref, zmax_ref, zmin_ref,
               ps_ref, ps2_ref):
    xc = x_ref[...]
    cbv = cb_ref[...]
    wa = wa_ref[...]
    zm = zn = acc = acc2 = None
    for j in range(_K):
        d = feat_ref[j] - xc
        z = lax.dot_general(d, wa, (((1,), (1,)), ((), ())),
                            preferred_element_type=jnp.float32) + cbv
        if j == 0:
            zm, zn, acc, acc2 = z, z, z, z * z
        else:
            zm = jnp.maximum(zm, z)
            zn = jnp.minimum(zn, z)
            acc = acc + z
            acc2 = acc2 + z * z
    zmax_ref[...] = zm
    zmin_ref[...] = zn
    ps_ref[0] = jnp.sum(acc, axis=0, keepdims=True)
    ps2_ref[0] = jnp.sum(acc2, axis=0, keepdims=True)


def _conv(feat, xf, cb, W, rb=256):
    BN, C = xf.shape
    O = W.shape[0]
    G = BN // rb
    out_shape = [
        jax.ShapeDtypeStruct((BN, O), jnp.float32),
        jax.ShapeDtypeStruct((BN, O), jnp.float32),
        jax.ShapeDtypeStruct((G, 1, O), jnp.float32),
        jax.ShapeDtypeStruct((G, 1, O), jnp.float32),
    ]
    zmax, zmin, ps, ps2 = pl.pallas_call(
        _conv_body,
        grid=(G,),
        in_specs=[
            pl.BlockSpec((_K, rb, C), lambda g: (0, g, 0)),
            pl.BlockSpec((rb, C), lambda g: (g, 0)),
            pl.BlockSpec((rb, O), lambda g: (g, 0)),
            pl.BlockSpec((O, C), lambda g: (0, 0)),
        ],
        out_specs=[
            pl.BlockSpec((rb, O), lambda g: (g, 0)),
            pl.BlockSpec((rb, O), lambda g: (g, 0)),
            pl.BlockSpec((1, 1, O), lambda g: (g, 0, 0)),
            pl.BlockSpec((1, 1, O), lambda g: (g, 0, 0)),
        ],
        out_shape=out_shape,
    )(feat.reshape(_K, BN, C), xf, cb, W)
    return zmax, zmin, ps.reshape(G, O), ps2.reshape(G, O)


def _fin_body(zmax_ref, zmin_ref, ps_ref, ps2_ref, g_ref, bta_ref, out_ref,
              *, cnt):
    s1 = jnp.sum(ps_ref[...], axis=0, keepdims=True)
    s2 = jnp.sum(ps2_ref[...], axis=0, keepdims=True)
    mean = s1 * (1.0 / cnt)
    var = s2 * (1.0 / cnt) - mean * mean
    inv = lax.rsqrt(var + _EPS)
    gam = g_ref[...]
    scale = gam * inv
    off = bta_ref[...] - mean * scale
    sel = jnp.where(gam >= 0.0, zmax_ref[...], zmin_ref[...])
    t = sel * scale + off
    out_ref[...] = jnp.where(t >= 0.0, t, _LEAKY * t)


def _fin(zmax, zmin, ps, ps2, g, bta, cnt, rb=512):
    BN, O = zmax.shape
    G = ps.shape[0]
    return pl.pallas_call(
        functools.partial(_fin_body, cnt=float(cnt)),
        grid=(BN // rb,),
        in_specs=[
            pl.BlockSpec((rb, O), lambda r: (r, 0)),
            pl.BlockSpec((rb, O), lambda r: (r, 0)),
            pl.BlockSpec((G, O), lambda r: (0, 0)),
            pl.BlockSpec((G, O), lambda r: (0, 0)),
            pl.BlockSpec((1, O), lambda r: (0, 0)),
            pl.BlockSpec((1, O), lambda r: (0, 0)),
        ],
        out_specs=pl.BlockSpec((rb, O), lambda r: (r, 0)),
        out_shape=jax.ShapeDtypeStruct((BN, O), jnp.float32),
    )(zmax, zmin, ps, ps2, g, bta)


def _layer(x, W, g, bta):
    B, N, C = x.shape
    O = W.shape[0]
    gidx, cb = _prep(x, W)
    gflatj = gidx.reshape(-1)
    xf = x.reshape(B * N, C)
    feat = _sc_gather(xf, gflatj)
    zmax, zmin, ps, ps2 = _conv(feat, xf, cb.reshape(B * N, O), W)
    out = _fin(zmax, zmin, ps, ps2, g.reshape(1, O), bta.reshape(1, O),
               B * N * _K)
    return out.reshape(B, N, O)


def kernel(interm_repr, W1, g1, b1, W2, g2, b2):
    x1 = _layer(interm_repr, W1, g1, b1)
    return _layer(x1, W2, g2, b2)


# speedup vs baseline: 18.3704x; 1.0970x over previous
import functools

import jax
import jax.numpy as jnp
from jax import lax
from jax.experimental import pallas as pl
from jax.experimental.pallas import tpu as pltpu
from jax.experimental.pallas import tpu_sc as plsc

_K = 5
_LEAKY = 0.2
_EPS = 1e-5


def _prep_body(xb_ref, wb_ref, gidx_ref, cb_ref, *, rb, n):
    b = pl.program_id(0)
    r = pl.program_id(1)
    xb = xb_ref[0]
    xr = xb_ref[0, pl.ds(r * rb, rb), :]
    x2b = jnp.sum(xb * xb, axis=1).reshape(1, n)
    x2r = jnp.sum(xr * xr, axis=1, keepdims=True)
    gram = lax.dot_general(xr, xb, (((1,), (1,)), ((), ())),
                           preferred_element_type=jnp.float32)
    dist = -(x2r + x2b - 2.0 * gram)
    colsf = lax.broadcasted_iota(jnp.int32, (rb, n), 1).astype(jnp.float32)
    base = b * n
    for j in range(_K):
        m = jnp.max(dist, axis=1, keepdims=True)
        fi = jnp.min(jnp.where(dist == m, colsf, float(n)), axis=1)
        gidx_ref[j, 0, 0, :] = fi.astype(jnp.int32) + base
        if j + 1 < _K:
            dist = jnp.where(colsf == fi[:, None], -jnp.inf, dist)
    cb_ref[0] = lax.dot_general(xr, wb_ref[...], (((1,), (1,)), ((), ())),
                                preferred_element_type=jnp.float32)


def _prep(x, W, rb=512):
    B, N, C = x.shape
    O = W.shape[0]
    out_shape = [
        jax.ShapeDtypeStruct((_K, B, 1, N), jnp.int32),
        jax.ShapeDtypeStruct((B, N, O), jnp.float32),
    ]
    return pl.pallas_call(
        functools.partial(_prep_body, rb=rb, n=N),
        grid=(B, N // rb),
        in_specs=[
            pl.BlockSpec((1, N, C), lambda b, r: (b, 0, 0)),
            pl.BlockSpec((O, C), lambda b, r: (0, 1)),
        ],
        out_specs=[
            pl.BlockSpec((_K, 1, 1, rb), lambda b, r: (0, b, 0, r)),
            pl.BlockSpec((1, rb, O), lambda b, r: (b, r, 0)),
        ],
        out_shape=out_shape,
    )(x, W)


def _sc_gather(xf, gidxj, cp=128):
    BN, C = xf.shape
    nw = 32
    P = BN // nw
    nstep = _K * (P // cp)
    mesh = plsc.VectorSubcoreMesh(core_axis_name="c", subcore_axis_name="s")

    nh = P // cp

    def body(xf_hbm, gidxj_hbm, feat_hbm, slab, rows0, rows1, rows2,
             gs0, gs1, gs2, ws0, ws1, ws2):
        wid = lax.axis_index("s") * 2 + lax.axis_index("c")
        base = wid * P
        rowsb = (rows0, rows1, rows2)
        gsem = (gs0, gs1, gs2)
        wsem = (ws0, ws1, ws2)

        for j in range(_K):
            pltpu.sync_copy(gidxj_hbm.at[pl.ds(j * BN + base, P)],
                            slab.at[pl.ds(j * P, P)])

        def off(t):
            j, h = divmod(t, nh)
            return j * BN + base + h * cp

        def idxsl(t):
            j, h = divmod(t, nh)
            return slab.at[pl.ds(j * P + h * cp, cp)]

        g = [None] * nstep
        w = [None] * nstep
        for s in range(min(3, nstep)):
            g[s] = pltpu.async_copy(xf_hbm.at[idxsl(s)], rowsb[s % 3],
                                    gsem[s % 3])
        for t in range(nstep):
            p = t % 3
            g[t].wait()
            w[t] = pltpu.async_copy(rowsb[p], feat_hbm.at[pl.ds(off(t), cp)],
                                    wsem[p])
            if t + 3 < nstep:
                w[t].wait()
                g[t + 3] = pltpu.async_copy(xf_hbm.at[idxsl(t + 3)], rowsb[p],
                                            gsem[p])
        for t in range(max(0, nstep - 3), nstep):
            w[t].wait()

    f = pl.kernel(
        body,
        out_type=jax.ShapeDtypeStruct((_K * BN, C), jnp.float32),
        mesh=mesh,
        scratch_types=[
            pltpu.VMEM((_K * P,), jnp.int32),
            pltpu.VMEM((cp, C), jnp.float32),
            pltpu.VMEM((cp, C), jnp.float32),
            pltpu.VMEM((cp, C), jnp.float32),
            pltpu.SemaphoreType.DMA,
            pltpu.SemaphoreType.DMA,
            pltpu.SemaphoreType.DMA,
            pltpu.SemaphoreType.DMA,
            pltpu.SemaphoreType.DMA,
            pltpu.SemaphoreType.DMA,
        ],
    )
    return f(xf, gidxj)


def _conv_body(feat_ref, x_ref, cb_ref, wa_ref, zmax_ref, zmin_ref,
               ps_ref, ps2_ref):
    xc = x_ref[...]
    cbv = cb_ref[...]
    wa = wa_ref[...]
    zm = zn = acc = acc2 = None
    for j in range(_K):
        d = feat_ref[j] - xc
        z = lax.dot_general(d, wa, (((1,), (1,)), ((), ())),
                            preferred_element_type=jnp.float32) + cbv
        if j == 0:
            zm, zn, acc, acc2 = z, z, z, z * z
        else:
            zm = jnp.maximum(zm, z)
            zn = jnp.minimum(zn, z)
            acc = acc + z
            acc2 = acc2 + z * z
    zmax_ref[...] = zm
    zmin_ref[...] = zn
    ps_ref[0] = jnp.sum(acc, axis=0, keepdims=True)
    ps2_ref[0] = jnp.sum(acc2, axis=0, keepdims=True)


def _conv(feat, xf, cb, W, rb=512):
    BN, C = xf.shape
    O = W.shape[0]
    G = BN // rb
    out_shape = [
        jax.ShapeDtypeStruct((BN, O), jnp.float32),
        jax.ShapeDtypeStruct((BN, O), jnp.float32),
        jax.ShapeDtypeStruct((G, 1, O), jnp.float32),
        jax.ShapeDtypeStruct((G, 1, O), jnp.float32),
    ]
    zmax, zmin, ps, ps2 = pl.pallas_call(
        _conv_body,
        grid=(G,),
        in_specs=[
            pl.BlockSpec((_K, rb, C), lambda g: (0, g, 0)),
            pl.BlockSpec((rb, C), lambda g: (g, 0)),
            pl.BlockSpec((rb, O), lambda g: (g, 0)),
            pl.BlockSpec((O, C), lambda g: (0, 0)),
        ],
        out_specs=[
            pl.BlockSpec((rb, O), lambda g: (g, 0)),
            pl.BlockSpec((rb, O), lambda g: (g, 0)),
            pl.BlockSpec((1, 1, O), lambda g: (g, 0, 0)),
            pl.BlockSpec((1, 1, O), lambda g: (g, 0, 0)),
        ],
        out_shape=out_shape,
    )(feat.reshape(_K, BN, C), xf, cb, W)
    return zmax, zmin, ps.reshape(G, O), ps2.reshape(G, O)


def _fin_body(zmax_ref, zmin_ref, ps_ref, ps2_ref, g_ref, bta_ref, out_ref,
              *, cnt):
    s1 = jnp.sum(ps_ref[...], axis=0, keepdims=True)
    s2 = jnp.sum(ps2_ref[...], axis=0, keepdims=True)
    mean = s1 * (1.0 / cnt)
    var = s2 * (1.0 / cnt) - mean * mean
    inv = lax.rsqrt(var + _EPS)
    gam = g_ref[...]
    scale = gam * inv
    off = bta_ref[...] - mean * scale
    sel = jnp.where(gam >= 0.0, zmax_ref[...], zmin_ref[...])
    t = sel * scale + off
    out_ref[...] = jnp.where(t >= 0.0, t, _LEAKY * t)


def _fin(zmax, zmin, ps, ps2, g, bta, cnt, rb=512):
    BN, O = zmax.shape
    G = ps.shape[0]
    return pl.pallas_call(
        functools.partial(_fin_body, cnt=float(cnt)),
        grid=(BN // rb,),
        in_specs=[
            pl.BlockSpec((rb, O), lambda r: (r, 0)),
            pl.BlockSpec((rb, O), lambda r: (r, 0)),
            pl.BlockSpec((G, O), lambda r: (0, 0)),
            pl.BlockSpec((G, O), lambda r: (0, 0)),
            pl.BlockSpec((1, O), lambda r: (0, 0)),
            pl.BlockSpec((1, O), lambda r: (0, 0)),
        ],
        out_specs=pl.BlockSpec((rb, O), lambda r: (r, 0)),
        out_shape=jax.ShapeDtypeStruct((BN, O), jnp.float32),
    )(zmax, zmin, ps, ps2, g, bta)


def _layer(x, W, g, bta):
    B, N, C = x.shape
    O = W.shape[0]
    gidx, cb = _prep(x, W)
    gflatj = gidx.reshape(-1)
    xf = x.reshape(B * N, C)
    feat = _sc_gather(xf, gflatj)
    zmax, zmin, ps, ps2 = _conv(feat, xf, cb.reshape(B * N, O), W)
    out = _fin(zmax, zmin, ps, ps2, g.reshape(1, O), bta.reshape(1, O),
               B * N * _K)
    return out.reshape(B, N, O)


def kernel(interm_repr, W1, g1, b1, W2, g2, b2):
    x1 = _layer(interm_repr, W1, g1, b1)
    return _layer(x1, W2, g2, b2)


# speedup vs baseline: 18.7077x; 1.0184x over previous
import functools

import jax
import jax.numpy as jnp
from jax import lax
from jax.experimental import pallas as pl
from jax.experimental.pallas import tpu as pltpu
from jax.experimental.pallas import tpu_sc as plsc

_K = 5
_LEAKY = 0.2
_EPS = 1e-5


def _prep_body(xb_ref, wb_ref, gidx_ref, cb_ref, *, rb, n):
    b = pl.program_id(0)
    r = pl.program_id(1)
    xb = xb_ref[0]
    xr = xb_ref[0, pl.ds(r * rb, rb), :]
    x2b = jnp.sum(xb * xb, axis=1).reshape(1, n)
    x2r = jnp.sum(xr * xr, axis=1, keepdims=True)
    gram = lax.dot_general(xr, xb, (((1,), (1,)), ((), ())),
                           preferred_element_type=jnp.float32)
    dist = -(x2r + x2b - 2.0 * gram)
    colsf = lax.broadcasted_iota(jnp.int32, (rb, n), 1).astype(jnp.float32)
    base = b * n
    for j in range(_K):
        m = jnp.max(dist, axis=1, keepdims=True)
        fi = jnp.min(jnp.where(dist == m, colsf, float(n)), axis=1)
        gidx_ref[j, 0, 0, :] = fi.astype(jnp.int32) + base
        if j + 1 < _K:
            dist = jnp.where(colsf == fi[:, None], -jnp.inf, dist)
    cb_ref[0] = lax.dot_general(xr, wb_ref[...], (((1,), (1,)), ((), ())),
                                preferred_element_type=jnp.float32)


def _prep(x, W, rb=512):
    B, N, C = x.shape
    O = W.shape[0]
    out_shape = [
        jax.ShapeDtypeStruct((_K, B, 1, N), jnp.int32),
        jax.ShapeDtypeStruct((B, N, O), jnp.float32),
    ]
    return pl.pallas_call(
        functools.partial(_prep_body, rb=rb, n=N),
        grid=(B, N // rb),
        in_specs=[
            pl.BlockSpec((1, N, C), lambda b, r: (b, 0, 0)),
            pl.BlockSpec((O, C), lambda b, r: (0, 1)),
        ],
        out_specs=[
            pl.BlockSpec((_K, 1, 1, rb), lambda b, r: (0, b, 0, r)),
            pl.BlockSpec((1, rb, O), lambda b, r: (b, r, 0)),
        ],
        out_shape=out_shape,
    )(x, W)


def _sc_gather(xf, gidxj, cp=128):
    BN, C = xf.shape
    nw = 32
    P = BN // nw
    nstep = _K * (P // cp)
    mesh = plsc.VectorSubcoreMesh(core_axis_name="c", subcore_axis_name="s")

    nh = P // cp

    def body(xf_hbm, gidxj_hbm, feat_hbm, slab, rows0, rows1, rows2,
             gs0, gs1, gs2, ws0, ws1, ws2):
        wid = lax.axis_index("s") * 2 + lax.axis_index("c")
        base = wid * P
        rowsb = (rows0, rows1, rows2)
        gsem = (gs0, gs1, gs2)
        wsem = (ws0, ws1, ws2)

        for j in range(_K):
            pltpu.sync_copy(gidxj_hbm.at[pl.ds(j * BN + base, P)],
                            slab.at[pl.ds(j * P, P)])

        def off(t):
            j, h = divmod(t, nh)
            return j * BN + base + h * cp

        def idxsl(t):
            j, h = divmod(t, nh)
            return slab.at[pl.ds(j * P + h * cp, cp)]

        g = [None] * nstep
        w = [None] * nstep
        for s in range(min(3, nstep)):
            g[s] = pltpu.async_copy(xf_hbm.at[idxsl(s)], rowsb[s % 3],
                                    gsem[s % 3])
        for t in range(nstep):
            p = t % 3
            g[t].wait()
            w[t] = pltpu.async_copy(rowsb[p], feat_hbm.at[pl.ds(off(t), cp)],
                                    wsem[p])
            if t + 3 < nstep:
                w[t].wait()
                g[t + 3] = pltpu.async_copy(xf_hbm.at[idxsl(t + 3)], rowsb[p],
                                            gsem[p])
        for t in range(max(0, nstep - 3), nstep):
            w[t].wait()

    f = pl.kernel(
        body,
        out_type=jax.ShapeDtypeStruct((_K * BN, C), jnp.float32),
        mesh=mesh,
        scratch_types=[
            pltpu.VMEM((_K * P,), jnp.int32),
            pltpu.VMEM((cp, C), jnp.float32),
            pltpu.VMEM((cp, C), jnp.float32),
            pltpu.VMEM((cp, C), jnp.float32),
            pltpu.SemaphoreType.DMA,
            pltpu.SemaphoreType.DMA,
            pltpu.SemaphoreType.DMA,
            pltpu.SemaphoreType.DMA,
            pltpu.SemaphoreType.DMA,
            pltpu.SemaphoreType.DMA,
        ],
    )
    return f(xf, gidxj)


def _conv_body(feat_ref, x_ref, cb_ref, wa_ref, zmax_ref, zmin_ref,
               ps_ref, ps2_ref):
    xc = x_ref[...]
    cbv = cb_ref[...]
    wa = wa_ref[...]
    zm = zn = acc = acc2 = None
    for j in range(_K):
        d = feat_ref[j] - xc
        z = lax.dot_general(d, wa, (((1,), (1,)), ((), ())),
                            preferred_element_type=jnp.float32) + cbv
        if j == 0:
            zm, zn, acc, acc2 = z, z, z, z * z
        else:
            zm = jnp.maximum(zm, z)
            zn = jnp.minimum(zn, z)
            acc = acc + z
            acc2 = acc2 + z * z
    zmax_ref[...] = zm
    zmin_ref[...] = zn
    ps_ref[0] = jnp.sum(acc, axis=0, keepdims=True)
    ps2_ref[0] = jnp.sum(acc2, axis=0, keepdims=True)


def _conv(feat, xf, cb, W, rb=1024):
    BN, C = xf.shape
    O = W.shape[0]
    G = BN // rb
    out_shape = [
        jax.ShapeDtypeStruct((BN, O), jnp.float32),
        jax.ShapeDtypeStruct((BN, O), jnp.float32),
        jax.ShapeDtypeStruct((G, 1, O), jnp.float32),
        jax.ShapeDtypeStruct((G, 1, O), jnp.float32),
    ]
    zmax, zmin, ps, ps2 = pl.pallas_call(
        _conv_body,
        grid=(G,),
        in_specs=[
            pl.BlockSpec((_K, rb, C), lambda g: (0, g, 0)),
            pl.BlockSpec((rb, C), lambda g: (g, 0)),
            pl.BlockSpec((rb, O), lambda g: (g, 0)),
            pl.BlockSpec((O, C), lambda g: (0, 0)),
        ],
        out_specs=[
            pl.BlockSpec((rb, O), lambda g: (g, 0)),
            pl.BlockSpec((rb, O), lambda g: (g, 0)),
            pl.BlockSpec((1, 1, O), lambda g: (g, 0, 0)),
            pl.BlockSpec((1, 1, O), lambda g: (g, 0, 0)),
        ],
        out_shape=out_shape,
    )(feat.reshape(_K, BN, C), xf, cb, W)
    return zmax, zmin, ps.reshape(G, O), ps2.reshape(G, O)


def _fin_body(zmax_ref, zmin_ref, ps_ref, ps2_ref, g_ref, bta_ref, out_ref,
              *, cnt):
    s1 = jnp.sum(ps_ref[...], axis=0, keepdims=True)
    s2 = jnp.sum(ps2_ref[...], axis=0, keepdims=True)
    mean = s1 * (1.0 / cnt)
    var = s2 * (1.0 / cnt) - mean * mean
    inv = lax.rsqrt(var + _EPS)
    gam = g_ref[...]
    scale = gam * inv
    off = bta_ref[...] - mean * scale
    sel = jnp.where(gam >= 0.0, zmax_ref[...], zmin_ref[...])
    t = sel * scale + off
    out_ref[...] = jnp.where(t >= 0.0, t, _LEAKY * t)


def _fin(zmax, zmin, ps, ps2, g, bta, cnt, rb=512):
    BN, O = zmax.shape
    G = ps.shape[0]
    return pl.pallas_call(
        functools.partial(_fin_body, cnt=float(cnt)),
        grid=(BN // rb,),
        in_specs=[
            pl.BlockSpec((rb, O), lambda r: (r, 0)),
            pl.BlockSpec((rb, O), lambda r: (r, 0)),
            pl.BlockSpec((G, O), lambda r: (0, 0)),
            pl.BlockSpec((G, O), lambda r: (0, 0)),
            pl.BlockSpec((1, O), lambda r: (0, 0)),
            pl.BlockSpec((1, O), lambda r: (0, 0)),
        ],
        out_specs=pl.BlockSpec((rb, O), lambda r: (r, 0)),
        out_shape=jax.ShapeDtypeStruct((BN, O), jnp.float32),
    )(zmax, zmin, ps, ps2, g, bta)


def _layer(x, W, g, bta):
    B, N, C = x.shape
    O = W.shape[0]
    gidx, cb = _prep(x, W)
    gflatj = gidx.reshape(-1)
    xf = x.reshape(B * N, C)
    feat = _sc_gather(xf, gflatj)
    zmax, zmin, ps, ps2 = _conv(feat, xf, cb.reshape(B * N, O), W)
    out = _fin(zmax, zmin, ps, ps2, g.reshape(1, O), bta.reshape(1, O),
               B * N * _K)
    return out.reshape(B, N, O)


def kernel(interm_repr, W1, g1, b1, W2, g2, b2):
    x1 = _layer(interm_repr, W1, g1, b1)
    return _layer(x1, W2, g2, b2)


# speedup vs baseline: 18.7838x; 1.0041x over previous
import functools

import jax
import jax.numpy as jnp
from jax import lax
from jax.experimental import pallas as pl
from jax.experimental.pallas import tpu as pltpu
from jax.experimental.pallas import tpu_sc as plsc

_K = 5
_LEAKY = 0.2
_EPS = 1e-5


def _prep_body(xb_ref, wb_ref, gidx_ref, cb_ref, *, rb, n):
    b = pl.program_id(0)
    r = pl.program_id(1)
    xb = xb_ref[0]
    xr = xb_ref[0, pl.ds(r * rb, rb), :]
    x2b = jnp.sum(xb * xb, axis=1).reshape(1, n)
    x2r = jnp.sum(xr * xr, axis=1, keepdims=True)
    gram = lax.dot_general(xr, xb, (((1,), (1,)), ((), ())),
                           preferred_element_type=jnp.float32)
    dist = -(x2r + x2b - 2.0 * gram)
    colsf = lax.broadcasted_iota(jnp.int32, (rb, n), 1).astype(jnp.float32)
    base = b * n
    for j in range(_K):
        m = jnp.max(dist, axis=1, keepdims=True)
        fi = jnp.min(jnp.where(dist == m, colsf, float(n)), axis=1)
        gidx_ref[j, 0, 0, :] = fi.astype(jnp.int32) + base
        if j + 1 < _K:
            dist = jnp.where(colsf == fi[:, None], -jnp.inf, dist)
    cb_ref[0] = lax.dot_general(xr, wb_ref[...], (((1,), (1,)), ((), ())),
                                preferred_element_type=jnp.float32)


def _prep(x, W, rb=512):
    B, N, C = x.shape
    O = W.shape[0]
    out_shape = [
        jax.ShapeDtypeStruct((_K, B, 1, N), jnp.int32),
        jax.ShapeDtypeStruct((B, N, O), jnp.float32),
    ]
    return pl.pallas_call(
        functools.partial(_prep_body, rb=rb, n=N),
        grid=(B, N // rb),
        in_specs=[
            pl.BlockSpec((1, N, C), lambda b, r: (b, 0, 0)),
            pl.BlockSpec((O, C), lambda b, r: (0, 1)),
        ],
        out_specs=[
            pl.BlockSpec((_K, 1, 1, rb), lambda b, r: (0, b, 0, r)),
            pl.BlockSpec((1, rb, O), lambda b, r: (b, r, 0)),
        ],
        out_shape=out_shape,
    )(x, W)


def _sc_gather(xf, gidxj, cp=128):
    BN, C = xf.shape
    nw = 32
    P = BN // nw
    nstep = _K * (P // cp)
    mesh = plsc.VectorSubcoreMesh(core_axis_name="c", subcore_axis_name="s")

    nh = P // cp

    def body(xf_hbm, gidxj_hbm, feat_hbm, slab, rows0, rows1, rows2,
             gs0, gs1, gs2, ws0, ws1, ws2):
        wid = lax.axis_index("s") * 2 + lax.axis_index("c")
        base = wid * P
        rowsb = (rows0, rows1, rows2)
        gsem = (gs0, gs1, gs2)
        wsem = (ws0, ws1, ws2)

        for j in range(_K):
            pltpu.sync_copy(gidxj_hbm.at[pl.ds(j * BN + base, P)],
                            slab.at[pl.ds(j * P, P)])

        def off(t):
            j, h = divmod(t, nh)
            return j * BN + base + h * cp

        def idxsl(t):
            j, h = divmod(t, nh)
            return slab.at[pl.ds(j * P + h * cp, cp)]

        g = [None] * nstep
        w = [None] * nstep
        for s in range(min(3, nstep)):
            g[s] = pltpu.async_copy(xf_hbm.at[idxsl(s)], rowsb[s % 3],
                                    gsem[s % 3])
        for t in range(nstep):
            p = t % 3
            g[t].wait()
            w[t] = pltpu.async_copy(rowsb[p], feat_hbm.at[pl.ds(off(t), cp)],
                                    wsem[p])
            if t + 3 < nstep:
                w[t].wait()
                g[t + 3] = pltpu.async_copy(xf_hbm.at[idxsl(t + 3)], rowsb[p],
                                            gsem[p])
        for t in range(max(0, nstep - 3), nstep):
            w[t].wait()

    f = pl.kernel(
        body,
        out_type=jax.ShapeDtypeStruct((_K * BN, C), jnp.float32),
        mesh=mesh,
        scratch_types=[
            pltpu.VMEM((_K * P,), jnp.int32),
            pltpu.VMEM((cp, C), jnp.float32),
            pltpu.VMEM((cp, C), jnp.float32),
            pltpu.VMEM((cp, C), jnp.float32),
            pltpu.SemaphoreType.DMA,
            pltpu.SemaphoreType.DMA,
            pltpu.SemaphoreType.DMA,
            pltpu.SemaphoreType.DMA,
            pltpu.SemaphoreType.DMA,
            pltpu.SemaphoreType.DMA,
        ],
    )
    return f(xf, gidxj)


def _conv_body(feat_ref, x_ref, cb_ref, wa_ref, zmax_ref, zmin_ref,
               ps_ref, ps2_ref):
    xc = x_ref[...]
    cbv = cb_ref[...]
    wa = wa_ref[...]
    zm = zn = acc = acc2 = None
    for j in range(_K):
        d = feat_ref[j] - xc
        z = lax.dot_general(d, wa, (((1,), (1,)), ((), ())),
                            preferred_element_type=jnp.float32) + cbv
        if j == 0:
            zm, zn, acc, acc2 = z, z, z, z * z
        else:
            zm = jnp.maximum(zm, z)
            zn = jnp.minimum(zn, z)
            acc = acc + z
            acc2 = acc2 + z * z
    zmax_ref[...] = zm
    zmin_ref[...] = zn
    ps_ref[0] = jnp.sum(acc, axis=0, keepdims=True)
    ps2_ref[0] = jnp.sum(acc2, axis=0, keepdims=True)


def _conv(feat, xf, cb, W, rb=512):
    BN, C = xf.shape
    O = W.shape[0]
    G = BN // rb
    out_shape = [
        jax.ShapeDtypeStruct((BN, O), jnp.float32),
        jax.ShapeDtypeStruct((BN, O), jnp.float32),
        jax.ShapeDtypeStruct((G, 1, O), jnp.float32),
        jax.ShapeDtypeStruct((G, 1, O), jnp.float32),
    ]
    zmax, zmin, ps, ps2 = pl.pallas_call(
        _conv_body,
        grid=(G,),
        in_specs=[
            pl.BlockSpec((_K, rb, C), lambda g: (0, g, 0)),
            pl.BlockSpec((rb, C), lambda g: (g, 0)),
            pl.BlockSpec((rb, O), lambda g: (g, 0)),
            pl.BlockSpec((O, C), lambda g: (0, 0)),
        ],
        out_specs=[
            pl.BlockSpec((rb, O), lambda g: (g, 0)),
            pl.BlockSpec((rb, O), lambda g: (g, 0)),
            pl.BlockSpec((1, 1, O), lambda g: (g, 0, 0)),
            pl.BlockSpec((1, 1, O), lambda g: (g, 0, 0)),
        ],
        out_shape=out_shape,
    )(feat.reshape(_K, BN, C), xf, cb, W)
    return zmax, zmin, ps.reshape(G, O), ps2.reshape(G, O)


def _fin_body(zmax_ref, zmin_ref, ps_ref, ps2_ref, g_ref, bta_ref, out_ref,
              *, cnt):
    s1 = jnp.sum(ps_ref[...], axis=0, keepdims=True)
    s2 = jnp.sum(ps2_ref[...], axis=0, keepdims=True)
    mean = s1 * (1.0 / cnt)
    var = s2 * (1.0 / cnt) - mean * mean
    inv = lax.rsqrt(var + _EPS)
    gam = g_ref[...]
    scale = gam * inv
    off = bta_ref[...] - mean * scale
    sel = jnp.where(gam >= 0.0, zmax_ref[...], zmin_ref[...])
    t = sel * scale + off
    out_ref[...] = jnp.where(t >= 0.0, t, _LEAKY * t)


def _fin(zmax, zmin, ps, ps2, g, bta, cnt, rb=512):
    BN, O = zmax.shape
    G = ps.shape[0]
    return pl.pallas_call(
        functools.partial(_fin_body, cnt=float(cnt)),
        grid=(BN // rb,),
        in_specs=[
            pl.BlockSpec((rb, O), lambda r: (r, 0)),
            pl.BlockSpec((rb, O), lambda r: (r, 0)),
            pl.BlockSpec((G, O), lambda r: (0, 0)),
            pl.BlockSpec((G, O), lambda r: (0, 0)),
            pl.BlockSpec((1, O), lambda r: (0, 0)),
            pl.BlockSpec((1, O), lambda r: (0, 0)),
        ],
        out_specs=pl.BlockSpec((rb, O), lambda r: (r, 0)),
        out_shape=jax.ShapeDtypeStruct((BN, O), jnp.float32),
    )(zmax, zmin, ps, ps2, g, bta)


def _layer(xs, W, g, bta, cnt):
    O = W.shape[0]
    pre = [_prep(xh, W) for xh in xs]
    feats = [_sc_gather(xh.reshape(-1, xh.shape[2]), p[0].reshape(-1))
             for xh, p in zip(xs, pre)]
    convs = [_conv(f, xh.reshape(-1, xh.shape[2]), p[1].reshape(-1, O), W)
             for f, xh, p in zip(feats, xs, pre)]
    ps = jnp.concatenate([c[2] for c in convs], 0)
    ps2 = jnp.concatenate([c[3] for c in convs], 0)
    outs = [_fin(c[0], c[1], ps, ps2, g.reshape(1, O), bta.reshape(1, O), cnt)
            for c in convs]
    return [o.reshape(xh.shape[0], xh.shape[1], O)
            for o, xh in zip(outs, xs)]


def kernel(interm_repr, W1, g1, b1, W2, g2, b2):
    x = interm_repr
    B, N, C = x.shape
    cnt = B * N * _K
    xs = [x[:B // 2], x[B // 2:]]
    x1s = _layer(xs, W1, g1, b1, cnt)
    x2s = _layer(x1s, W2, g2, b2, cnt)
    return jnp.concatenate(x2s, 0)
